# baseline (speedup 1.0000x reference)
import functools

import jax
import jax.numpy as jnp
from jax import lax
from jax.experimental import pallas as pl
from jax.experimental.pallas import tpu as pltpu

N_NODES = 10000
N_HYPEREDGES = 5000
NNZ = 320000
E_INNER = 640000
D = 128
NUM_CLASSES = 10
NUM_GRAPHS = 16
NUM_LAYERS = 2
INNER_LAYERS = 2


def _mlp_body(x_ref, agg_ref, w1_ref, b1_ref, w2_ref, b2_ref, o_ref):
    h = x_ref[...] + agg_ref[...]
    h = jnp.maximum(jnp.dot(h, w1_ref[...], preferred_element_type=jnp.float32)
                    + b1_ref[...], 0.0)
    o_ref[...] = jnp.maximum(jnp.dot(h, w2_ref[...], preferred_element_type=jnp.float32)
                             + b2_ref[...], 0.0)


@jax.jit
def _gin_mlp(x, agg, w1, b1, w2, b2):
    n = x.shape[0]
    blk = 2000
    grid = (n // blk,)
    row_spec = pl.BlockSpec((blk, D), lambda i: (i, 0))
    w_spec = pl.BlockSpec((D, D), lambda i: (0, 0))
    b_spec = pl.BlockSpec((1, D), lambda i: (0, 0))
    return pl.pallas_call(
        _mlp_body,
        grid=grid,
        in_specs=[row_spec, row_spec, w_spec, b_spec, w_spec, b_spec],
        out_specs=row_spec,
        out_shape=jax.ShapeDtypeStruct((n, D), jnp.float32),
    )(x, agg, w1, b1.reshape(1, D), w2, b2.reshape(1, D))


def _emb_body(x_ref, w_ref, b_ref, o_ref):
    o_ref[...] = jnp.dot(x_ref[...], w_ref[...],
                         preferred_element_type=jnp.float32) + b_ref[...]


@jax.jit
def _emb(x, w, b):
    n = x.shape[0]
    blk = 2000
    return pl.pallas_call(
        _emb_body,
        grid=(n // blk,),
        in_specs=[pl.BlockSpec((blk, D), lambda i: (i, 0)),
                  pl.BlockSpec((D, D), lambda i: (0, 0)),
                  pl.BlockSpec((1, D), lambda i: (0, 0))],
        out_specs=pl.BlockSpec((blk, D), lambda i: (i, 0)),
        out_shape=jax.ShapeDtypeStruct((n, D), jnp.float32),
    )(x, w, b.reshape(1, D))


def kernel(x_N, W_emb, b_emb, gin_W1, gin_b1, gin_W2, gin_b2, W_pred, b_pred,
           ori_node_idx, node2edge, ori_edge_idx, edge2node,
           edge_index_N, edge_index_E, batch):
    node_x = _emb(x_N, W_emb, b_emb)
    xs = [node_x]
    for l in range(NUM_LAYERS):
        _nx = node_x[ori_node_idx]
        for c in range(INNER_LAYERS):
            idx = l * 4 + c
            agg = jax.ops.segment_sum(_nx[edge_index_N[0]], edge_index_N[1],
                                      num_segments=NNZ)
            _nx = _gin_mlp(_nx, agg, gin_W1[idx], gin_b1[idx],
                           gin_W2[idx], gin_b2[idx])
        edge_x = jax.nn.relu(jax.ops.segment_sum(_nx, node2edge,
                                                 num_segments=N_HYPEREDGES))
        _ex = edge_x[ori_edge_idx]
        for c in range(INNER_LAYERS):
            idx = l * 4 + 2 + c
            agg = jax.ops.segment_sum(_ex[edge_index_E[0]], edge_index_E[1],
                                      num_segments=NNZ)
            _ex = _gin_mlp(_ex, agg, gin_W1[idx], gin_b1[idx],
                           gin_W2[idx], gin_b2[idx])
        node_x = jax.nn.relu(jax.ops.segment_sum(_ex, edge2node,
                                                 num_segments=N_NODES))
        xs.append(node_x)
    score = jnp.zeros((NUM_GRAPHS, NUM_CLASSES), jnp.float32)
    for i, x in enumerate(xs):
        pooled = jax.ops.segment_sum(x[ori_node_idx], batch,
                                     num_segments=NUM_GRAPHS)
        score = score + pooled @ W_pred[i] + b_pred[i]
    return score
